# baseline (speedup 1.0000x reference)
import functools

import jax
import jax.numpy as jnp
from jax import lax
from jax.experimental import pallas as pl
from jax.experimental.pallas import tpu as pltpu
from jax.experimental.pallas import tpu_sc as plsc

N = 10000
E = 320000
D_IN = 128
HIDDEN = 128
CLS = 40

NP = 10240
D2P = 128

NC, NS = 2, 16
NW = NC * NS
CH = 88
ITERS = 116
EPT = ITERS * CH
EP = NW * EPT
RPS = NP // NS
NSL = 4


CH_D = 128
ITERS_D = 80
EP_D = NW * ITERS_D * CH_D


@functools.cache
def _get_sc_degree():
    mesh = plsc.VectorSubcoreMesh(core_axis_name="c", subcore_axis_name="s",
                                  num_cores=NC, num_subcores=NS)
    return functools.partial(
        pl.kernel,
        out_type=jax.ShapeDtypeStruct((NC * NP,), jnp.float32),
        mesh=mesh,
        scratch_types=[
            pltpu.VMEM((ITERS_D, CH_D), jnp.int32),
            pltpu.VMEM((CH_D,), jnp.float32),
            pltpu.VMEM((RPS,), jnp.float32),
            pltpu.VMEM_SHARED((NP,), jnp.float32),
        ],
    )(_sc_degree_body)


def _sc_degree_body(dst2_hbm, out_hbm, idx_v, ones_v, zbuf_v, acc_sh):
    c = lax.axis_index("c")
    s = lax.axis_index("s")
    wid = s * NC + c
    for i in range(RPS // 16):
        zbuf_v[pl.ds(i * 16, 16)] = jnp.zeros((16,), jnp.float32)
    for i in range(CH_D // 16):
        ones_v[pl.ds(i * 16, 16)] = jnp.ones((16,), jnp.float32)
    pltpu.sync_copy(dst2_hbm.at[pl.ds(wid * ITERS_D, ITERS_D)], idx_v)
    pltpu.sync_copy(zbuf_v, acc_sh.at[pl.ds(s * RPS, RPS)])
    plsc.subcore_barrier()

    def body(it, carry):
        pltpu.sync_copy(ones_v, acc_sh.at[idx_v.at[it]], add=True)
        return carry

    lax.fori_loop(0, ITERS_D, body, 0)
    plsc.subcore_barrier()
    pltpu.sync_copy(acc_sh.at[pl.ds(s * RPS, RPS)],
                    out_hbm.at[pl.ds(c * NP + s * RPS, RPS)])


@functools.cache
def _make_sc_agg(D):
    mesh = plsc.VectorSubcoreMesh(core_axis_name="c", subcore_axis_name="s",
                                  num_cores=NC, num_subcores=NS)

    @functools.partial(
        pl.kernel,
        out_type=jax.ShapeDtypeStruct((NC * NP, D), jnp.float32),
        mesh=mesh,
        scratch_types=(
            [pltpu.VMEM((CH,), jnp.int32) for _ in range(NSL)]
            + [pltpu.VMEM((CH,), jnp.int32) for _ in range(NSL)]
            + [
                pltpu.VMEM((NSL, CH, D), jnp.float32),
                pltpu.VMEM_SHARED((NP, D), jnp.float32),
            ]
            + [pltpu.SemaphoreType.DMA] * (4 * NSL)
        ),
    )
    def _agg(y_hbm, src1_hbm, dst1_hbm, out_hbm, *refs):
        sidx = refs[0:NSL]
        didx = refs[NSL:2 * NSL]
        rows_v = refs[2 * NSL]
        acc_sh = refs[2 * NSL + 1]
        sems = refs[2 * NSL + 2:]
        isem = sems[0:NSL]
        dsem = sems[NSL:2 * NSL]
        gsem = sems[2 * NSL:3 * NSL]
        ssem = sems[3 * NSL:4 * NSL]
        c = lax.axis_index("c")
        s = lax.axis_index("s")
        wid = s * NC + c
        ebase = wid * EPT

        def _src_at(it):
            return src1_hbm.at[pl.ds(ebase + it * CH, CH)]

        def _dst_at(it):
            return dst1_hbm.at[pl.ds(ebase + it * CH, CH)]

        def zrow(i, carry):
            for j in range(D // 16):
                rows_v[0, i, pl.ds(j * 16, 16)] = jnp.zeros((16,), jnp.float32)
            return carry

        lax.fori_loop(0, CH, zrow, 0)
        for k in range(RPS // CH):
            pltpu.sync_copy(rows_v.at[0],
                            acc_sh.at[pl.ds(s * RPS + k * CH, CH)])
        pltpu.sync_copy(rows_v.at[0, pl.ds(0, RPS - (RPS // CH) * CH)],
                        acc_sh.at[pl.ds(s * RPS + (RPS // CH) * CH,
                                        RPS - (RPS // CH) * CH)])
        for b in range(NSL - 1):
            pltpu.sync_copy(_src_at(b), sidx[b])
            pltpu.async_copy(y_hbm.at[sidx[b]], rows_v.at[b], gsem[b])
        pltpu.async_copy(_src_at(NSL - 1), sidx[NSL - 1], isem[NSL - 1])
        for b in range(NSL - 1):
            pltpu.async_copy(_dst_at(b), didx[b], dsem[b])
        plsc.subcore_barrier()

        def outer(o, carry):
            for b in range(NSL):
                it = o * NSL + b
                p = (b + NSL - 1) % NSL

                pltpu.make_async_copy(y_hbm.at[sidx[b]], rows_v.at[b],
                                      gsem[b]).wait()

                @pl.when(it + NSL < ITERS)
                def _():
                    pltpu.async_copy(_src_at(it + NSL), sidx[b], isem[b])

                pltpu.make_async_copy(_dst_at(it), didx[b],
                                      dsem[b]).wait()
                pltpu.async_copy(rows_v.at[b], acc_sh.at[didx[b]], ssem[b],
                                 add=True)

                @pl.when(it >= 1)
                def _():
                    pltpu.make_async_copy(rows_v.at[p], acc_sh.at[didx[p]],
                                          ssem[p]).wait()

                @pl.when(it + NSL - 1 < ITERS)
                def _():
                    pltpu.async_copy(_dst_at(it + NSL - 1), didx[p], dsem[p])
                    pltpu.make_async_copy(_src_at(it + NSL - 1), sidx[p],
                                          isem[p]).wait()
                    pltpu.async_copy(y_hbm.at[sidx[p]], rows_v.at[p], gsem[p])
            return carry

        lax.fori_loop(0, ITERS // NSL, outer, 0)
        lastb = (ITERS - 1) % NSL
        pltpu.make_async_copy(rows_v.at[lastb], acc_sh.at[didx[lastb]],
                              ssem[lastb]).wait()
        plsc.subcore_barrier()
        pltpu.sync_copy(acc_sh.at[pl.ds(s * RPS, RPS)],
                        out_hbm.at[pl.ds(c * NP + s * RPS, RPS)])

    return _agg


BR = 1024


def _tc1_body(x_ref, w1_ref, d0_ref, d1_ref, xw_ref, y1_ref, dis_ref):
    xw = jnp.dot(x_ref[...], w1_ref[...], preferred_element_type=jnp.float32)
    deg = d0_ref[...] + d1_ref[...] + 1.0
    dis = lax.rsqrt(deg)
    xw_ref[...] = xw
    y1_ref[...] = xw * dis
    dis_ref[...] = dis


_tc1 = pl.pallas_call(
    _tc1_body,
    grid=(NP // BR,),
    in_specs=[
        pl.BlockSpec((BR, D_IN), lambda i: (i, 0)),
        pl.BlockSpec((D_IN, HIDDEN), lambda i: (0, 0)),
        pl.BlockSpec((BR, 1), lambda i: (i, 0)),
        pl.BlockSpec((BR, 1), lambda i: (i, 0)),
    ],
    out_specs=[
        pl.BlockSpec((BR, HIDDEN), lambda i: (i, 0)),
        pl.BlockSpec((BR, HIDDEN), lambda i: (i, 0)),
        pl.BlockSpec((BR, 1), lambda i: (i, 0)),
    ],
    out_shape=[
        jax.ShapeDtypeStruct((NP, HIDDEN), jnp.float32),
        jax.ShapeDtypeStruct((NP, HIDDEN), jnp.float32),
        jax.ShapeDtypeStruct((NP, 1), jnp.float32),
    ],
)


def _tc2_body(a0_ref, a1_ref, xw_ref, dis_ref, b1_ref, w2_ref, hw_ref,
              y2_ref):
    dis = dis_ref[...]
    h = (a0_ref[...] + a1_ref[...]) * dis + xw_ref[...] * (dis * dis)
    h = jnp.maximum(h + b1_ref[...], 0.0)
    hw = jnp.dot(h, w2_ref[...], preferred_element_type=jnp.float32)
    hw_ref[...] = hw
    y2_ref[...] = hw * dis


_tc2 = pl.pallas_call(
    _tc2_body,
    grid=(NP // BR,),
    in_specs=[
        pl.BlockSpec((BR, HIDDEN), lambda i: (i, 0)),
        pl.BlockSpec((BR, HIDDEN), lambda i: (i, 0)),
        pl.BlockSpec((BR, HIDDEN), lambda i: (i, 0)),
        pl.BlockSpec((BR, 1), lambda i: (i, 0)),
        pl.BlockSpec((1, HIDDEN), lambda i: (0, 0)),
        pl.BlockSpec((HIDDEN, D2P), lambda i: (0, 0)),
    ],
    out_specs=[
        pl.BlockSpec((BR, D2P), lambda i: (i, 0)),
        pl.BlockSpec((BR, D2P), lambda i: (i, 0)),
    ],
    out_shape=[
        jax.ShapeDtypeStruct((NP, D2P), jnp.float32),
        jax.ShapeDtypeStruct((NP, D2P), jnp.float32),
    ],
)


def _tc3_body(a0_ref, a1_ref, hw_ref, dis_ref, b2_ref, o_ref):
    dis = dis_ref[...]
    o_ref[...] = ((a0_ref[...] + a1_ref[...]) * dis
                  + hw_ref[...] * (dis * dis) + b2_ref[...])


_tc3 = pl.pallas_call(
    _tc3_body,
    grid=(NP // BR,),
    in_specs=[
        pl.BlockSpec((BR, D2P), lambda i: (i, 0)),
        pl.BlockSpec((BR, D2P), lambda i: (i, 0)),
        pl.BlockSpec((BR, D2P), lambda i: (i, 0)),
        pl.BlockSpec((BR, 1), lambda i: (i, 0)),
        pl.BlockSpec((1, D2P), lambda i: (0, 0)),
    ],
    out_specs=pl.BlockSpec((BR, D2P), lambda i: (i, 0)),
    out_shape=jax.ShapeDtypeStruct((NP, D2P), jnp.float32),
)


def kernel(x, edge_index, W1, b1, W2, b2):
    src1 = jnp.concatenate(
        [edge_index[0], jnp.zeros((EP - E,), edge_index.dtype)])
    dst1 = jnp.concatenate(
        [edge_index[1], jnp.full((EP - E,), N, edge_index.dtype)])
    dst2 = jnp.concatenate(
        [edge_index[1], jnp.full((EP_D - E,), N, edge_index.dtype)]
    ).reshape(EP_D // CH_D, CH_D)
    x_p = jnp.zeros((NP, D_IN), x.dtype).at[:N].set(x)
    W2_p = jnp.zeros((HIDDEN, D2P), W2.dtype).at[:, :CLS].set(W2)
    b2_p = jnp.zeros((1, D2P), b2.dtype).at[0, :CLS].set(b2)

    degp = _get_sc_degree()(dst2)
    d0 = degp[:NP].reshape(NP, 1)
    d1 = degp[NP:].reshape(NP, 1)

    xw, y1, dis = _tc1(x_p, W1, d0, d1)
    acc1 = _make_sc_agg(HIDDEN)(y1, src1, dst1)
    hw, y2 = _tc2(acc1[:NP], acc1[NP:], xw, dis, b1.reshape(1, HIDDEN), W2_p)
    acc2 = _make_sc_agg(D2P)(y2, src1, dst1)
    out_p = _tc3(acc2[:NP], acc2[NP:], hw, dis, b2_p)
    return out_p[:N, :CLS]


# speedup vs baseline: 2.8024x; 2.8024x over previous
import functools

import jax
import jax.numpy as jnp
from jax import lax
from jax.experimental import pallas as pl
from jax.experimental.pallas import tpu as pltpu
from jax.experimental.pallas import tpu_sc as plsc

N = 10000
E = 320000
D_IN = 128
HIDDEN = 128
CLS = 40

NP = 10240
D2P = 128

NC, NS = 2, 16
NW = NC * NS
CH = 88
ITERS = 116
EPT = ITERS * CH
EP = NW * EPT
RPS = NP // NS
NSL = 4


CH_D = 128
ITERS_D = 80
EP_D = NW * ITERS_D * CH_D


@functools.cache
def _get_sc_degree():
    mesh = plsc.VectorSubcoreMesh(core_axis_name="c", subcore_axis_name="s",
                                  num_cores=NC, num_subcores=NS)
    return functools.partial(
        pl.kernel,
        out_type=jax.ShapeDtypeStruct((NC * NP,), jnp.float32),
        mesh=mesh,
        scratch_types=[
            pltpu.VMEM((ITERS_D, CH_D), jnp.int32),
            pltpu.VMEM((CH_D,), jnp.float32),
            pltpu.VMEM((RPS,), jnp.float32),
            pltpu.VMEM_SHARED((NP,), jnp.float32),
        ],
    )(_sc_degree_body)


def _sc_degree_body(dst2_hbm, out_hbm, idx_v, ones_v, zbuf_v, acc_sh):
    c = lax.axis_index("c")
    s = lax.axis_index("s")
    wid = s * NC + c
    for i in range(RPS // 16):
        zbuf_v[pl.ds(i * 16, 16)] = jnp.zeros((16,), jnp.float32)
    for i in range(CH_D // 16):
        ones_v[pl.ds(i * 16, 16)] = jnp.ones((16,), jnp.float32)
    pltpu.sync_copy(dst2_hbm.at[pl.ds(wid * ITERS_D, ITERS_D)], idx_v)
    pltpu.sync_copy(zbuf_v, acc_sh.at[pl.ds(s * RPS, RPS)])
    plsc.subcore_barrier()

    def body(it, carry):
        pltpu.sync_copy(ones_v, acc_sh.at[idx_v.at[it]], add=True)
        return carry

    lax.fori_loop(0, ITERS_D, body, 0)
    plsc.subcore_barrier()
    pltpu.sync_copy(acc_sh.at[pl.ds(s * RPS, RPS)],
                    out_hbm.at[pl.ds(c * NP + s * RPS, RPS)])


@functools.cache
def _make_sc_agg(D):
    mesh = plsc.VectorSubcoreMesh(core_axis_name="c", subcore_axis_name="s",
                                  num_cores=NC, num_subcores=NS)

    @functools.partial(
        pl.kernel,
        out_type=jax.ShapeDtypeStruct((NC * NP, D), jnp.float32),
        mesh=mesh,
        scratch_types=(
            [pltpu.VMEM((CH,), jnp.int32) for _ in range(NSL)]
            + [pltpu.VMEM((CH,), jnp.int32) for _ in range(NSL)]
            + [
                pltpu.VMEM((NSL, CH, D), jnp.float32),
                pltpu.VMEM_SHARED((NP, D), jnp.float32),
            ]
            + [pltpu.SemaphoreType.DMA] * (4 * NSL)
        ),
    )
    def _agg(y_hbm, src1_hbm, dst1_hbm, out_hbm, *refs):
        sidx = refs[0:NSL]
        didx = refs[NSL:2 * NSL]
        rows_v = refs[2 * NSL]
        acc_sh = refs[2 * NSL + 1]
        sems = refs[2 * NSL + 2:]
        isem = sems[0:NSL]
        dsem = sems[NSL:2 * NSL]
        gsem = sems[2 * NSL:3 * NSL]
        ssem = sems[3 * NSL:4 * NSL]
        c = lax.axis_index("c")
        s = lax.axis_index("s")
        wid = s * NC + c
        ebase = wid * EPT

        def _src_at(it):
            return src1_hbm.at[pl.ds(ebase + it * CH, CH)]

        def _dst_at(it):
            return dst1_hbm.at[pl.ds(ebase + it * CH, CH)]

        def zrow(i, carry):
            for j in range(D // 16):
                rows_v[0, i, pl.ds(j * 16, 16)] = jnp.zeros((16,), jnp.float32)
            return carry

        lax.fori_loop(0, CH, zrow, 0)
        for k in range(RPS // CH):
            pltpu.sync_copy(rows_v.at[0],
                            acc_sh.at[pl.ds(s * RPS + k * CH, CH)])
        pltpu.sync_copy(rows_v.at[0, pl.ds(0, RPS - (RPS // CH) * CH)],
                        acc_sh.at[pl.ds(s * RPS + (RPS // CH) * CH,
                                        RPS - (RPS // CH) * CH)])
        for b in range(NSL - 1):
            pltpu.sync_copy(_src_at(b), sidx[b])
            pltpu.async_copy(y_hbm.at[sidx[b]], rows_v.at[b], gsem[b])
        pltpu.async_copy(_src_at(NSL - 1), sidx[NSL - 1], isem[NSL - 1])
        for b in range(NSL - 1):
            pltpu.async_copy(_dst_at(b), didx[b], dsem[b])
        plsc.subcore_barrier()

        def outer(o, carry):
            for b in range(NSL):
                it = o * NSL + b
                p = (b + NSL - 1) % NSL

                pltpu.make_async_copy(y_hbm.at[sidx[b]], rows_v.at[b],
                                      gsem[b]).wait()

                @pl.when(it + NSL < ITERS)
                def _():
                    pltpu.async_copy(_src_at(it + NSL), sidx[b], isem[b])

                pltpu.make_async_copy(_dst_at(it), didx[b],
                                      dsem[b]).wait()
                pltpu.async_copy(rows_v.at[b], acc_sh.at[didx[b]], ssem[b],
                                 add=True)

                @pl.when(it >= 1)
                def _():
                    pltpu.make_async_copy(rows_v.at[p], acc_sh.at[didx[p]],
                                          ssem[p]).wait()

                @pl.when(it + NSL - 1 < ITERS)
                def _():
                    pltpu.async_copy(_dst_at(it + NSL - 1), didx[p], dsem[p])
                    pltpu.make_async_copy(_src_at(it + NSL - 1), sidx[p],
                                          isem[p]).wait()
                    pltpu.async_copy(y_hbm.at[sidx[p]], rows_v.at[p], gsem[p])
            return carry

        lax.fori_loop(0, ITERS // NSL, outer, 0)
        lastb = (ITERS - 1) % NSL
        pltpu.make_async_copy(rows_v.at[lastb], acc_sh.at[didx[lastb]],
                              ssem[lastb]).wait()
        plsc.subcore_barrier()
        pltpu.sync_copy(acc_sh.at[pl.ds(s * RPS, RPS)],
                        out_hbm.at[pl.ds(c * NP + s * RPS, RPS)])

    return _agg


BR = 1024


def _tc1_body(x_ref, w1_ref, d0_ref, d1_ref, xw_ref, y1_ref, dis_ref):
    xw = jnp.dot(x_ref[...], w1_ref[...], preferred_element_type=jnp.float32)
    deg = d0_ref[...] + d1_ref[...] + 1.0
    dis = lax.rsqrt(deg)
    xw_ref[...] = xw
    y1_ref[...] = xw * dis
    dis_ref[...] = dis


_tc1 = pl.pallas_call(
    _tc1_body,
    grid=(NP // BR,),
    in_specs=[
        pl.BlockSpec((BR, D_IN), lambda i: (i, 0)),
        pl.BlockSpec((D_IN, HIDDEN), lambda i: (0, 0)),
        pl.BlockSpec((BR, 1), lambda i: (i, 0)),
        pl.BlockSpec((BR, 1), lambda i: (i, 0)),
    ],
    out_specs=[
        pl.BlockSpec((BR, HIDDEN), lambda i: (i, 0)),
        pl.BlockSpec((BR, HIDDEN), lambda i: (i, 0)),
        pl.BlockSpec((BR, 1), lambda i: (i, 0)),
    ],
    out_shape=[
        jax.ShapeDtypeStruct((NP, HIDDEN), jnp.float32),
        jax.ShapeDtypeStruct((NP, HIDDEN), jnp.float32),
        jax.ShapeDtypeStruct((NP, 1), jnp.float32),
    ],
)


def _tc2_body(a0_ref, a1_ref, xw_ref, dis_ref, b1_ref, w2_ref, hw_ref,
              y2_ref):
    dis = dis_ref[...]
    h = (a0_ref[...] + a1_ref[...]) * dis + xw_ref[...] * (dis * dis)
    h = jnp.maximum(h + b1_ref[...], 0.0)
    hw = jnp.dot(h, w2_ref[...], preferred_element_type=jnp.float32)
    hw_ref[...] = hw
    y2_ref[...] = hw * dis


_tc2 = pl.pallas_call(
    _tc2_body,
    grid=(NP // BR,),
    in_specs=[
        pl.BlockSpec((BR, HIDDEN), lambda i: (i, 0)),
        pl.BlockSpec((BR, HIDDEN), lambda i: (i, 0)),
        pl.BlockSpec((BR, HIDDEN), lambda i: (i, 0)),
        pl.BlockSpec((BR, 1), lambda i: (i, 0)),
        pl.BlockSpec((1, HIDDEN), lambda i: (0, 0)),
        pl.BlockSpec((HIDDEN, D2P), lambda i: (0, 0)),
    ],
    out_specs=[
        pl.BlockSpec((BR, D2P), lambda i: (i, 0)),
        pl.BlockSpec((BR, D2P), lambda i: (i, 0)),
    ],
    out_shape=[
        jax.ShapeDtypeStruct((NP, D2P), jnp.float32),
        jax.ShapeDtypeStruct((NP, D2P), jnp.float32),
    ],
)


def _tc3_body(a0_ref, a1_ref, hw_ref, dis_ref, b2_ref, o_ref):
    dis = dis_ref[...]
    o_ref[...] = ((a0_ref[...] + a1_ref[...]) * dis
                  + hw_ref[...] * (dis * dis) + b2_ref[...])


_tc3 = pl.pallas_call(
    _tc3_body,
    grid=(NP // BR,),
    in_specs=[
        pl.BlockSpec((BR, D2P), lambda i: (i, 0)),
        pl.BlockSpec((BR, D2P), lambda i: (i, 0)),
        pl.BlockSpec((BR, D2P), lambda i: (i, 0)),
        pl.BlockSpec((BR, 1), lambda i: (i, 0)),
        pl.BlockSpec((1, D2P), lambda i: (0, 0)),
    ],
    out_specs=pl.BlockSpec((BR, D2P), lambda i: (i, 0)),
    out_shape=jax.ShapeDtypeStruct((NP, D2P), jnp.float32),
)


def kernel(x, edge_index, W1, b1, W2, b2):
    pad_a = jnp.arange(EP - E, dtype=edge_index.dtype)
    pad_d = jnp.arange(EP_D - E, dtype=edge_index.dtype)
    src1 = jnp.concatenate([edge_index[0], pad_a % N])
    dst1 = jnp.concatenate([edge_index[1], N + pad_a % (NP - N)])
    dst2 = jnp.concatenate(
        [edge_index[1], N + pad_d % (NP - N)]
    ).reshape(EP_D // CH_D, CH_D)
    x_p = jnp.zeros((NP, D_IN), x.dtype).at[:N].set(x)
    W2_p = jnp.zeros((HIDDEN, D2P), W2.dtype).at[:, :CLS].set(W2)
    b2_p = jnp.zeros((1, D2P), b2.dtype).at[0, :CLS].set(b2)

    degp = _get_sc_degree()(dst2)
    d0 = degp[:NP].reshape(NP, 1)
    d1 = degp[NP:].reshape(NP, 1)

    xw, y1, dis = _tc1(x_p, W1, d0, d1)
    acc1 = _make_sc_agg(HIDDEN)(y1, src1, dst1)
    hw, y2 = _tc2(acc1[:NP], acc1[NP:], xw, dis, b1.reshape(1, HIDDEN), W2_p)
    acc2 = _make_sc_agg(D2P)(y2, src1, dst1)
    out_p = _tc3(acc2[:NP], acc2[NP:], hw, dis, b2_p)
    return out_p[:N, :CLS]


# speedup vs baseline: 2.8343x; 1.0114x over previous
import functools

import jax
import jax.numpy as jnp
from jax import lax
from jax.experimental import pallas as pl
from jax.experimental.pallas import tpu as pltpu
from jax.experimental.pallas import tpu_sc as plsc

N = 10000
E = 320000
D_IN = 128
HIDDEN = 128
CLS = 40

NP = 10240
D2P = 128

NC, NS = 2, 16
NW = NC * NS
CH = 88
ITERS = 116
EPT = ITERS * CH
EP = NW * EPT
RPS = NP // NS
NSL = 4


ONES = 96


@functools.cache
def _get_sc_degree():
    mesh = plsc.VectorSubcoreMesh(core_axis_name="c", subcore_axis_name="s",
                                  num_cores=NC, num_subcores=NS)
    return functools.partial(
        pl.kernel,
        out_type=jax.ShapeDtypeStruct((NC * NP,), jnp.float32),
        mesh=mesh,
        scratch_types=[
            pltpu.VMEM((CH,), jnp.int32),
            pltpu.VMEM((CH,), jnp.int32),
            pltpu.VMEM((ONES,), jnp.float32),
            pltpu.VMEM((RPS,), jnp.float32),
            pltpu.VMEM_SHARED((NP,), jnp.float32),
            pltpu.SemaphoreType.DMA,
            pltpu.SemaphoreType.DMA,
        ],
    )(_sc_degree_body)


def _sc_degree_body(dst1_hbm, out_hbm, didx0, didx1, ones_v, zbuf_v, acc_sh,
                    dsem0, dsem1):
    c = lax.axis_index("c")
    s = lax.axis_index("s")
    wid = s * NC + c
    didx = (didx0, didx1)
    dsem = (dsem0, dsem1)
    ebase = wid * EPT

    def _dst_at(it):
        return dst1_hbm.at[pl.ds(ebase + it * CH, CH)]

    for i in range(RPS // 16):
        zbuf_v[pl.ds(i * 16, 16)] = jnp.zeros((16,), jnp.float32)
    for i in range(ONES // 16):
        ones_v[pl.ds(i * 16, 16)] = jnp.ones((16,), jnp.float32)
    pltpu.sync_copy(_dst_at(0), didx0)
    pltpu.async_copy(_dst_at(1), didx1, dsem1)
    pltpu.sync_copy(zbuf_v, acc_sh.at[pl.ds(s * RPS, RPS)])
    plsc.subcore_barrier()

    def outer(o, carry):
        for b in (0, 1):
            it = o * 2 + b

            @pl.when(it >= 1)
            def _():
                pltpu.make_async_copy(_dst_at(it), didx[b], dsem[b]).wait()

            pltpu.sync_copy(ones_v.at[pl.ds(0, CH)], acc_sh.at[didx[b]],
                            add=True)

            @pl.when(it + 2 < ITERS)
            def _():
                pltpu.async_copy(_dst_at(it + 2), didx[b], dsem[b])
        return carry

    lax.fori_loop(0, ITERS // 2, outer, 0)
    plsc.subcore_barrier()
    pltpu.sync_copy(acc_sh.at[pl.ds(s * RPS, RPS)],
                    out_hbm.at[pl.ds(c * NP + s * RPS, RPS)])


@functools.cache
def _make_sc_agg(D):
    mesh = plsc.VectorSubcoreMesh(core_axis_name="c", subcore_axis_name="s",
                                  num_cores=NC, num_subcores=NS)

    @functools.partial(
        pl.kernel,
        out_type=jax.ShapeDtypeStruct((NC * NP, D), jnp.float32),
        mesh=mesh,
        scratch_types=(
            [pltpu.VMEM((CH,), jnp.int32) for _ in range(NSL)]
            + [pltpu.VMEM((CH,), jnp.int32) for _ in range(NSL)]
            + [
                pltpu.VMEM((NSL, CH, D), jnp.float32),
                pltpu.VMEM_SHARED((NP, D), jnp.float32),
            ]
            + [pltpu.SemaphoreType.DMA] * (4 * NSL)
        ),
    )
    def _agg(y_hbm, src1_hbm, dst1_hbm, out_hbm, *refs):
        sidx = refs[0:NSL]
        didx = refs[NSL:2 * NSL]
        rows_v = refs[2 * NSL]
        acc_sh = refs[2 * NSL + 1]
        sems = refs[2 * NSL + 2:]
        isem = sems[0:NSL]
        dsem = sems[NSL:2 * NSL]
        gsem = sems[2 * NSL:3 * NSL]
        ssem = sems[3 * NSL:4 * NSL]
        c = lax.axis_index("c")
        s = lax.axis_index("s")
        wid = s * NC + c
        ebase = wid * EPT

        def _src_at(it):
            return src1_hbm.at[pl.ds(ebase + it * CH, CH)]

        def _dst_at(it):
            return dst1_hbm.at[pl.ds(ebase + it * CH, CH)]

        def zrow(i, carry):
            for j in range(D // 16):
                rows_v[0, i, pl.ds(j * 16, 16)] = jnp.zeros((16,), jnp.float32)
            return carry

        lax.fori_loop(0, CH, zrow, 0)
        for k in range(RPS // CH):
            pltpu.sync_copy(rows_v.at[0],
                            acc_sh.at[pl.ds(s * RPS + k * CH, CH)])
        pltpu.sync_copy(rows_v.at[0, pl.ds(0, RPS - (RPS // CH) * CH)],
                        acc_sh.at[pl.ds(s * RPS + (RPS // CH) * CH,
                                        RPS - (RPS // CH) * CH)])
        for b in range(NSL - 1):
            pltpu.sync_copy(_src_at(b), sidx[b])
            pltpu.async_copy(y_hbm.at[sidx[b]], rows_v.at[b], gsem[b])
        pltpu.async_copy(_src_at(NSL - 1), sidx[NSL - 1], isem[NSL - 1])
        for b in range(NSL - 1):
            pltpu.async_copy(_dst_at(b), didx[b], dsem[b])
        plsc.subcore_barrier()

        def outer(o, carry):
            for b in range(NSL):
                it = o * NSL + b
                p = (b + NSL - 1) % NSL

                pltpu.make_async_copy(y_hbm.at[sidx[b]], rows_v.at[b],
                                      gsem[b]).wait()

                @pl.when(it + NSL < ITERS)
                def _():
                    pltpu.async_copy(_src_at(it + NSL), sidx[b], isem[b])

                pltpu.make_async_copy(_dst_at(it), didx[b],
                                      dsem[b]).wait()
                pltpu.async_copy(rows_v.at[b], acc_sh.at[didx[b]], ssem[b],
                                 add=True)

                @pl.when(it >= 1)
                def _():
                    pltpu.make_async_copy(rows_v.at[p], acc_sh.at[didx[p]],
                                          ssem[p]).wait()

                @pl.when(it + NSL - 1 < ITERS)
                def _():
                    pltpu.async_copy(_dst_at(it + NSL - 1), didx[p], dsem[p])
                    pltpu.make_async_copy(_src_at(it + NSL - 1), sidx[p],
                                          isem[p]).wait()
                    pltpu.async_copy(y_hbm.at[sidx[p]], rows_v.at[p], gsem[p])
            return carry

        lax.fori_loop(0, ITERS // NSL, outer, 0)
        lastb = (ITERS - 1) % NSL
        pltpu.make_async_copy(rows_v.at[lastb], acc_sh.at[didx[lastb]],
                              ssem[lastb]).wait()
        plsc.subcore_barrier()
        pltpu.sync_copy(acc_sh.at[pl.ds(s * RPS, RPS)],
                        out_hbm.at[pl.ds(c * NP + s * RPS, RPS)])

    return _agg


BR = 1024


def _tc1_body(x_ref, w1_ref, d0_ref, d1_ref, y1_ref, dis_ref):
    xw = jnp.dot(x_ref[...], w1_ref[...], preferred_element_type=jnp.float32)
    deg = d0_ref[...] + d1_ref[...] + 1.0
    dis = lax.rsqrt(deg)
    y1_ref[...] = xw * dis
    dis_ref[...] = dis


_tc1 = pl.pallas_call(
    _tc1_body,
    grid=(NP // BR,),
    in_specs=[
        pl.BlockSpec((BR, D_IN), lambda i: (i, 0)),
        pl.BlockSpec((D_IN, HIDDEN), lambda i: (0, 0)),
        pl.BlockSpec((BR, 1), lambda i: (i, 0)),
        pl.BlockSpec((BR, 1), lambda i: (i, 0)),
    ],
    out_specs=[
        pl.BlockSpec((BR, HIDDEN), lambda i: (i, 0)),
        pl.BlockSpec((BR, 1), lambda i: (i, 0)),
    ],
    out_shape=[
        jax.ShapeDtypeStruct((NP, HIDDEN), jnp.float32),
        jax.ShapeDtypeStruct((NP, 1), jnp.float32),
    ],
)


def _tc2_body(a0_ref, a1_ref, y1_ref, dis_ref, b1_ref, w2_ref, y2_ref):
    dis = dis_ref[...]
    h = (a0_ref[...] + a1_ref[...] + y1_ref[...]) * dis
    h = jnp.maximum(h + b1_ref[...], 0.0)
    hw = jnp.dot(h, w2_ref[...], preferred_element_type=jnp.float32)
    y2_ref[...] = hw * dis


_tc2 = pl.pallas_call(
    _tc2_body,
    grid=(NP // BR,),
    in_specs=[
        pl.BlockSpec((BR, HIDDEN), lambda i: (i, 0)),
        pl.BlockSpec((BR, HIDDEN), lambda i: (i + NP // BR, 0)),
        pl.BlockSpec((BR, HIDDEN), lambda i: (i, 0)),
        pl.BlockSpec((BR, 1), lambda i: (i, 0)),
        pl.BlockSpec((1, HIDDEN), lambda i: (0, 0)),
        pl.BlockSpec((HIDDEN, D2P), lambda i: (0, 0)),
    ],
    out_specs=pl.BlockSpec((BR, D2P), lambda i: (i, 0)),
    out_shape=jax.ShapeDtypeStruct((NP, D2P), jnp.float32),
)


def _tc3_body(a0_ref, a1_ref, y2_ref, dis_ref, b2_ref, o_ref):
    dis = dis_ref[...]
    out = (a0_ref[...] + a1_ref[...] + y2_ref[...]) * dis
    o_ref[...] = out[:, :CLS] + b2_ref[...]


_tc3 = pl.pallas_call(
    _tc3_body,
    grid=(NP // BR,),
    in_specs=[
        pl.BlockSpec((BR, D2P), lambda i: (i, 0)),
        pl.BlockSpec((BR, D2P), lambda i: (i + NP // BR, 0)),
        pl.BlockSpec((BR, D2P), lambda i: (i, 0)),
        pl.BlockSpec((BR, 1), lambda i: (i, 0)),
        pl.BlockSpec((1, CLS), lambda i: (0, 0)),
    ],
    out_specs=pl.BlockSpec((BR, CLS), lambda i: (i, 0)),
    out_shape=jax.ShapeDtypeStruct((NP, CLS), jnp.float32),
)


def kernel(x, edge_index, W1, b1, W2, b2):
    pad_a = jnp.arange(EP - E, dtype=edge_index.dtype)
    pad_blk = jnp.stack([pad_a % N, N + pad_a % (NP - N)])
    ei_p = jnp.concatenate([edge_index, pad_blk], axis=1)
    src1 = ei_p[0]
    dst1 = ei_p[1]
    x_p = jnp.zeros((NP, D_IN), x.dtype).at[:N].set(x)
    W2_p = jnp.zeros((HIDDEN, D2P), W2.dtype).at[:, :CLS].set(W2)

    degp = _get_sc_degree()(dst1)
    d0 = degp[:NP].reshape(NP, 1)
    d1 = degp[NP:].reshape(NP, 1)

    y1, dis = _tc1(x_p, W1, d0, d1)
    acc1 = _make_sc_agg(HIDDEN)(y1, src1, dst1)
    y2 = _tc2(acc1, acc1, y1, dis, b1.reshape(1, HIDDEN), W2_p)
    acc2 = _make_sc_agg(D2P)(y2, src1, dst1)
    out_p = _tc3(acc2, acc2, y2, dis, b2.reshape(1, CLS))
    return out_p[:N]


# speedup vs baseline: 2.9821x; 1.0521x over previous
import functools

import jax
import jax.numpy as jnp
from jax import lax
from jax.experimental import pallas as pl
from jax.experimental.pallas import tpu as pltpu
from jax.experimental.pallas import tpu_sc as plsc

N = 10000
E = 320000
D_IN = 128
HIDDEN = 128
CLS = 40

NP = 10240
D2P = 128

NC, NS = 2, 16
NW = NC * NS
CH = 88
ITERS = 116
EPT = ITERS * CH
EP = NW * EPT
RPS = NP // NS
NSL = 4


ONES = 96


@functools.cache
def _get_sc_degree():
    mesh = plsc.VectorSubcoreMesh(core_axis_name="c", subcore_axis_name="s",
                                  num_cores=NC, num_subcores=NS)
    return functools.partial(
        pl.kernel,
        out_type=jax.ShapeDtypeStruct((NC * NP,), jnp.float32),
        mesh=mesh,
        scratch_types=(
            [pltpu.VMEM((CH,), jnp.int32) for _ in range(NSL)]
            + [
                pltpu.VMEM((ONES,), jnp.float32),
                pltpu.VMEM((RPS,), jnp.float32),
                pltpu.VMEM_SHARED((NP,), jnp.float32),
            ]
            + [pltpu.SemaphoreType.DMA] * (2 * NSL)
        ),
    )(_sc_degree_body)


def _sc_degree_body(dst1_hbm, out_hbm, *refs):
    didx = refs[0:NSL]
    ones_v, zbuf_v, acc_sh = refs[NSL:NSL + 3]
    dsem = refs[NSL + 3:NSL + 3 + NSL]
    ssem = refs[NSL + 3 + NSL:]
    c = lax.axis_index("c")
    s = lax.axis_index("s")
    wid = s * NC + c
    ebase = wid * EPT

    def _dst_at(it):
        return dst1_hbm.at[pl.ds(ebase + it * CH, CH)]

    for i in range(RPS // 16):
        zbuf_v[pl.ds(i * 16, 16)] = jnp.zeros((16,), jnp.float32)
    for i in range(ONES // 16):
        ones_v[pl.ds(i * 16, 16)] = jnp.ones((16,), jnp.float32)
    for b in range(NSL - 1):
        pltpu.async_copy(_dst_at(b), didx[b], dsem[b])
    pltpu.sync_copy(zbuf_v, acc_sh.at[pl.ds(s * RPS, RPS)])
    plsc.subcore_barrier()

    def outer(o, carry):
        for b in range(NSL):
            it = o * NSL + b
            p = (b + NSL - 1) % NSL

            pltpu.make_async_copy(_dst_at(it), didx[b],
                                  dsem[b]).wait()
            pltpu.async_copy(ones_v.at[pl.ds(0, CH)], acc_sh.at[didx[b]],
                             ssem[b], add=True)

            @pl.when(it >= 1)
            def _():
                pltpu.make_async_copy(ones_v.at[pl.ds(0, CH)],
                                      acc_sh.at[didx[p]], ssem[p]).wait()

            @pl.when(it + NSL - 1 < ITERS)
            def _():
                pltpu.async_copy(_dst_at(it + NSL - 1), didx[p], dsem[p])
        return carry

    lax.fori_loop(0, ITERS // NSL, outer, 0)
    lastb = (ITERS - 1) % NSL
    pltpu.make_async_copy(ones_v.at[pl.ds(0, CH)], acc_sh.at[didx[lastb]],
                          ssem[lastb]).wait()
    plsc.subcore_barrier()
    pltpu.sync_copy(acc_sh.at[pl.ds(s * RPS, RPS)],
                    out_hbm.at[pl.ds(c * NP + s * RPS, RPS)])


@functools.cache
def _make_sc_agg(D):
    mesh = plsc.VectorSubcoreMesh(core_axis_name="c", subcore_axis_name="s",
                                  num_cores=NC, num_subcores=NS)

    @functools.partial(
        pl.kernel,
        out_type=jax.ShapeDtypeStruct((NC * NP, D), jnp.float32),
        mesh=mesh,
        scratch_types=(
            [pltpu.VMEM((CH,), jnp.int32) for _ in range(NSL)]
            + [pltpu.VMEM((CH,), jnp.int32) for _ in range(NSL)]
            + [
                pltpu.VMEM((NSL, CH, D), jnp.float32),
                pltpu.VMEM_SHARED((NP, D), jnp.float32),
            ]
            + [pltpu.SemaphoreType.DMA] * (4 * NSL)
        ),
    )
    def _agg(y_hbm, src1_hbm, dst1_hbm, out_hbm, *refs):
        sidx = refs[0:NSL]
        didx = refs[NSL:2 * NSL]
        rows_v = refs[2 * NSL]
        acc_sh = refs[2 * NSL + 1]
        sems = refs[2 * NSL + 2:]
        isem = sems[0:NSL]
        dsem = sems[NSL:2 * NSL]
        gsem = sems[2 * NSL:3 * NSL]
        ssem = sems[3 * NSL:4 * NSL]
        c = lax.axis_index("c")
        s = lax.axis_index("s")
        wid = s * NC + c
        ebase = wid * EPT

        def _src_at(it):
            return src1_hbm.at[pl.ds(ebase + it * CH, CH)]

        def _dst_at(it):
            return dst1_hbm.at[pl.ds(ebase + it * CH, CH)]

        def zrow(i, carry):
            for j in range(D // 16):
                rows_v[0, i, pl.ds(j * 16, 16)] = jnp.zeros((16,), jnp.float32)
            return carry

        lax.fori_loop(0, CH, zrow, 0)
        for k in range(RPS // CH):
            pltpu.sync_copy(rows_v.at[0],
                            acc_sh.at[pl.ds(s * RPS + k * CH, CH)])
        pltpu.sync_copy(rows_v.at[0, pl.ds(0, RPS - (RPS // CH) * CH)],
                        acc_sh.at[pl.ds(s * RPS + (RPS // CH) * CH,
                                        RPS - (RPS // CH) * CH)])
        for b in range(NSL - 1):
            pltpu.sync_copy(_src_at(b), sidx[b])
            pltpu.async_copy(y_hbm.at[sidx[b]], rows_v.at[b], gsem[b])
        pltpu.async_copy(_src_at(NSL - 1), sidx[NSL - 1], isem[NSL - 1])
        for b in range(NSL - 1):
            pltpu.async_copy(_dst_at(b), didx[b], dsem[b])
        plsc.subcore_barrier()

        def outer(o, carry):
            for b in range(NSL):
                it = o * NSL + b
                p = (b + NSL - 1) % NSL

                pltpu.make_async_copy(y_hbm.at[sidx[b]], rows_v.at[b],
                                      gsem[b]).wait()

                @pl.when(it + NSL < ITERS)
                def _():
                    pltpu.async_copy(_src_at(it + NSL), sidx[b], isem[b])

                pltpu.make_async_copy(_dst_at(it), didx[b],
                                      dsem[b]).wait()
                pltpu.async_copy(rows_v.at[b], acc_sh.at[didx[b]], ssem[b],
                                 add=True)

                @pl.when(it >= 1)
                def _():
                    pltpu.make_async_copy(rows_v.at[p], acc_sh.at[didx[p]],
                                          ssem[p]).wait()

                @pl.when(it + NSL - 1 < ITERS)
                def _():
                    pltpu.async_copy(_dst_at(it + NSL - 1), didx[p], dsem[p])
                    pltpu.make_async_copy(_src_at(it + NSL - 1), sidx[p],
                                          isem[p]).wait()
                    pltpu.async_copy(y_hbm.at[sidx[p]], rows_v.at[p], gsem[p])
            return carry

        lax.fori_loop(0, ITERS // NSL, outer, 0)
        lastb = (ITERS - 1) % NSL
        pltpu.make_async_copy(rows_v.at[lastb], acc_sh.at[didx[lastb]],
                              ssem[lastb]).wait()
        plsc.subcore_barrier()
        pltpu.sync_copy(acc_sh.at[pl.ds(s * RPS, RPS)],
                        out_hbm.at[pl.ds(c * NP + s * RPS, RPS)])

    return _agg


BR = 1024


def _tc1_body(x_ref, w1_ref, d0_ref, d1_ref, y1_ref, dis_ref):
    xw = jnp.dot(x_ref[...], w1_ref[...], preferred_element_type=jnp.float32)
    deg = d0_ref[...] + d1_ref[...] + 1.0
    dis = lax.rsqrt(deg)
    y1_ref[...] = xw * dis
    dis_ref[...] = dis


_tc1 = pl.pallas_call(
    _tc1_body,
    grid=(NP // BR,),
    in_specs=[
        pl.BlockSpec((BR, D_IN), lambda i: (i, 0)),
        pl.BlockSpec((D_IN, HIDDEN), lambda i: (0, 0)),
        pl.BlockSpec((BR, 1), lambda i: (i, 0)),
        pl.BlockSpec((BR, 1), lambda i: (i, 0)),
    ],
    out_specs=[
        pl.BlockSpec((BR, HIDDEN), lambda i: (i, 0)),
        pl.BlockSpec((BR, 1), lambda i: (i, 0)),
    ],
    out_shape=[
        jax.ShapeDtypeStruct((NP, HIDDEN), jnp.float32),
        jax.ShapeDtypeStruct((NP, 1), jnp.float32),
    ],
)


def _tc2_body(a0_ref, a1_ref, y1_ref, dis_ref, b1_ref, w2_ref, y2_ref):
    dis = dis_ref[...]
    h = (a0_ref[...] + a1_ref[...] + y1_ref[...]) * dis
    h = jnp.maximum(h + b1_ref[...], 0.0)
    hw = jnp.dot(h, w2_ref[...], preferred_element_type=jnp.float32)
    y2_ref[...] = hw * dis


_tc2 = pl.pallas_call(
    _tc2_body,
    grid=(NP // BR,),
    in_specs=[
        pl.BlockSpec((BR, HIDDEN), lambda i: (i, 0)),
        pl.BlockSpec((BR, HIDDEN), lambda i: (i + NP // BR, 0)),
        pl.BlockSpec((BR, HIDDEN), lambda i: (i, 0)),
        pl.BlockSpec((BR, 1), lambda i: (i, 0)),
        pl.BlockSpec((1, HIDDEN), lambda i: (0, 0)),
        pl.BlockSpec((HIDDEN, D2P), lambda i: (0, 0)),
    ],
    out_specs=pl.BlockSpec((BR, D2P), lambda i: (i, 0)),
    out_shape=jax.ShapeDtypeStruct((NP, D2P), jnp.float32),
)


def _tc3_body(a0_ref, a1_ref, y2_ref, dis_ref, b2_ref, o_ref):
    dis = dis_ref[...]
    out = (a0_ref[...] + a1_ref[...] + y2_ref[...]) * dis
    o_ref[...] = out[:, :CLS] + b2_ref[...]


_tc3 = pl.pallas_call(
    _tc3_body,
    grid=(NP // BR,),
    in_specs=[
        pl.BlockSpec((BR, D2P), lambda i: (i, 0)),
        pl.BlockSpec((BR, D2P), lambda i: (i + NP // BR, 0)),
        pl.BlockSpec((BR, D2P), lambda i: (i, 0)),
        pl.BlockSpec((BR, 1), lambda i: (i, 0)),
        pl.BlockSpec((1, CLS), lambda i: (0, 0)),
    ],
    out_specs=pl.BlockSpec((BR, CLS), lambda i: (i, 0)),
    out_shape=jax.ShapeDtypeStruct((N, CLS), jnp.float32),
)


def kernel(x, edge_index, W1, b1, W2, b2):
    pad_a = jnp.arange(EP - E, dtype=edge_index.dtype)
    pad_blk = jnp.stack([pad_a % N, N + pad_a % (NP - N)])
    ei_p = jnp.concatenate([edge_index, pad_blk], axis=1)
    src1 = ei_p[0]
    dst1 = ei_p[1]
    W2_p = jnp.zeros((HIDDEN, D2P), W2.dtype).at[:, :CLS].set(W2)

    degp = _get_sc_degree()(dst1)
    d0 = degp[:NP].reshape(NP, 1)
    d1 = degp[NP:].reshape(NP, 1)

    y1, dis = _tc1(x, W1, d0, d1)
    acc1 = _make_sc_agg(HIDDEN)(y1, src1, dst1)
    y2 = _tc2(acc1, acc1, y1, dis, b1.reshape(1, HIDDEN), W2_p)
    acc2 = _make_sc_agg(D2P)(y2, src1, dst1)
    return _tc3(acc2, acc2, y2, dis, b2.reshape(1, CLS))
